# Initial kernel scaffold; baseline (speedup 1.0000x reference)
import jax
import jax.numpy as jnp
from jax.experimental import pallas as pl


def kernel(x, weight):
    raise NotImplementedError("write your pallas kernel here")


# speedup vs baseline: 3.1093x; 3.1093x over previous
import functools

import jax
import jax.numpy as jnp
from jax.experimental import pallas as pl

T = 16384
DIM = 2048
NE = 64
TOPK = 8
NG = 8
TOPK_G = 4
SCALE = 2.5
GSZ = NE // NG

NEG = -1e30


def _router_body(x_ref, wt_ref, w_out_ref, i_out_ref):
    x = x_ref[...]
    wt = wt_ref[...]
    logits = jnp.dot(x, wt, preferred_element_type=jnp.float32)
    s = jax.nn.sigmoid(logits)

    lane = jax.lax.broadcasted_iota(jnp.int32, s.shape, 1)
    gid = lane // GSZ

    gmax = jnp.full(s.shape, NEG, jnp.float32)
    for g in range(NG):
        in_g = gid == g
        mg = jnp.max(jnp.where(in_g, s, NEG), axis=1, keepdims=True)
        gmax = jnp.where(in_g, mg, gmax)

    work = gmax
    keep = jnp.zeros(s.shape, jnp.bool_)
    for _ in range(TOPK_G):
        m = jnp.max(work, axis=1, keepdims=True)
        mgid = jnp.min(jnp.where(work >= m, gid, 127), axis=1, keepdims=True)
        chosen = gid == mgid
        keep = jnp.logical_or(keep, chosen)
        work = jnp.where(chosen, NEG, work)

    sm = jnp.where(keep, s, NEG)
    vals, idxs = [], []
    for _ in range(TOPK):
        m = jnp.max(sm, axis=1, keepdims=True)
        mi = jnp.min(jnp.where(sm >= m, lane, 127), axis=1, keepdims=True)
        vals.append(m)
        idxs.append(mi)
        sm = jnp.where(lane == mi, NEG, sm)

    v = jnp.concatenate(vals, axis=1)
    w_out_ref[...] = v * (SCALE / jnp.sum(v, axis=1, keepdims=True))
    i_out_ref[...] = jnp.concatenate(idxs, axis=1)


@functools.partial(jax.jit, static_argnames=("rows",))
def _route(x, wt, rows=512):
    grid = (T // rows,)
    return pl.pallas_call(
        _router_body,
        grid=grid,
        in_specs=[
            pl.BlockSpec((rows, DIM), lambda i: (i, 0)),
            pl.BlockSpec((DIM, NE), lambda i: (0, 0)),
        ],
        out_specs=[
            pl.BlockSpec((rows, TOPK), lambda i: (i, 0)),
            pl.BlockSpec((rows, TOPK), lambda i: (i, 0)),
        ],
        out_shape=[
            jax.ShapeDtypeStruct((T, TOPK), jnp.float32),
            jax.ShapeDtypeStruct((T, TOPK), jnp.int32),
        ],
    )(x, wt)


def kernel(x, weight):
    w, i = _route(x, weight.T)
    return w, i


# speedup vs baseline: 8.0006x; 2.5731x over previous
import functools

import jax
import jax.numpy as jnp
from jax.experimental import pallas as pl

T = 16384
DIM = 2048
NE = 64
TOPK = 8
NG = 8
TOPK_G = 4
SCALE = 2.5
GSZ = NE // NG

NEG = -1e30


def _router_body(x_ref, wt_ref, w_out_ref, i_out_ref):
    r = x_ref.shape[0]
    logits = jnp.dot(x_ref[...], wt_ref[...],
                     preferred_element_type=jnp.float32)
    st = jax.nn.sigmoid(logits.T)

    row = jax.lax.broadcasted_iota(jnp.int32, (NE, r), 0)
    grow = jax.lax.broadcasted_iota(jnp.int32, (NG, r), 0)

    gmax = jnp.concatenate(
        [jnp.max(st[g * GSZ:(g + 1) * GSZ, :], axis=0, keepdims=True)
         for g in range(NG)], axis=0)

    work = gmax
    keep = jnp.zeros((NE, r), jnp.bool_)
    for _ in range(TOPK_G):
        m = jnp.max(work, axis=0, keepdims=True)
        mg = jnp.min(jnp.where(work >= m, grow, 127),
                     axis=0, keepdims=True)
        keep = jnp.logical_or(keep, (row // GSZ) == mg)
        work = jnp.where(grow == mg, NEG, work)

    sm = jnp.where(keep, st, NEG)
    vals, idxs = [], []
    for _ in range(TOPK):
        m = jnp.max(sm, axis=0, keepdims=True)
        mi = jnp.min(jnp.where(sm >= m, row, 127),
                     axis=0, keepdims=True)
        vals.append(m)
        idxs.append(mi)
        sm = jnp.where(row == mi, NEG, sm)

    v = jnp.concatenate(vals, axis=0)
    w_out_ref[...] = v * (SCALE / jnp.sum(v, axis=0, keepdims=True))
    i_out_ref[...] = jnp.concatenate(idxs, axis=0)


@functools.partial(jax.jit, static_argnames=("rows",))
def _route(x, wt, rows=512):
    grid = (T // rows,)
    return pl.pallas_call(
        _router_body,
        grid=grid,
        in_specs=[
            pl.BlockSpec((rows, DIM), lambda i: (i, 0)),
            pl.BlockSpec((DIM, NE), lambda i: (0, 0)),
        ],
        out_specs=[
            pl.BlockSpec((TOPK, rows), lambda i: (0, i)),
            pl.BlockSpec((TOPK, rows), lambda i: (0, i)),
        ],
        out_shape=[
            jax.ShapeDtypeStruct((TOPK, T), jnp.float32),
            jax.ShapeDtypeStruct((TOPK, T), jnp.int32),
        ],
    )(x, wt)


def kernel(x, weight):
    w, i = _route(x, weight.T)
    return w.T, i.T


# speedup vs baseline: 9.6646x; 1.2080x over previous
import functools

import jax
import jax.numpy as jnp
from jax.experimental import pallas as pl

T = 16384
DIM = 2048
NE = 64
TOPK = 8
NG = 8
TOPK_G = 4
SCALE = 2.5
GSZ = NE // NG

NEG = -1e30


def _router_body(x_ref, wt_ref, w_out_ref, i_out_ref):
    r = x_ref.shape[0]
    logits = jnp.dot(x_ref[...], wt_ref[...],
                     preferred_element_type=jnp.float32)
    st = jax.nn.sigmoid(logits.T)

    row = jax.lax.broadcasted_iota(jnp.int32, (NE, r), 0)
    grow = jax.lax.broadcasted_iota(jnp.int32, (NG, r), 0)

    gmax = jnp.concatenate(
        [jnp.max(st[g * GSZ:(g + 1) * GSZ, :], axis=0, keepdims=True)
         for g in range(NG)], axis=0)

    work = gmax
    keep = jnp.zeros((NE, r), jnp.bool_)
    for _ in range(TOPK_G):
        m = jnp.max(work, axis=0, keepdims=True)
        mg = jnp.min(jnp.where(work >= m, grow, 127),
                     axis=0, keepdims=True)
        keep = jnp.logical_or(keep, (row // GSZ) == mg)
        work = jnp.where(grow == mg, NEG, work)

    sm = jnp.where(keep, st, NEG)
    vals, idxs = [], []
    for _ in range(TOPK):
        m = jnp.max(sm, axis=0, keepdims=True)
        mi = jnp.min(jnp.where(sm >= m, row, 127),
                     axis=0, keepdims=True)
        vals.append(m)
        idxs.append(mi)
        sm = jnp.where(row == mi, NEG, sm)

    v = jnp.concatenate(vals, axis=0)
    w_out_ref[...] = v * (SCALE / jnp.sum(v, axis=0, keepdims=True))
    i_out_ref[...] = jnp.concatenate(idxs, axis=0)


@functools.partial(jax.jit, static_argnames=("rows",))
def _route(x, wt, rows=1024):
    grid = (T // rows,)
    return pl.pallas_call(
        _router_body,
        grid=grid,
        in_specs=[
            pl.BlockSpec((rows, DIM), lambda i: (i, 0)),
            pl.BlockSpec((DIM, NE), lambda i: (0, 0)),
        ],
        out_specs=[
            pl.BlockSpec((TOPK, rows), lambda i: (0, i)),
            pl.BlockSpec((TOPK, rows), lambda i: (0, i)),
        ],
        out_shape=[
            jax.ShapeDtypeStruct((TOPK, T), jnp.float32),
            jax.ShapeDtypeStruct((TOPK, T), jnp.int32),
        ],
    )(x, wt)


def kernel(x, weight):
    w, i = _route(x, weight.T)
    return w.T, i.T


# speedup vs baseline: 9.9214x; 1.0266x over previous
import functools

import jax
import jax.numpy as jnp
from jax.experimental import pallas as pl

T = 16384
DIM = 2048
NE = 64
TOPK = 8
NG = 8
TOPK_G = 4
SCALE = 2.5
GSZ = NE // NG

NEG = -1e30


def _router_body(x_ref, wt_ref, w_out_ref, i_out_ref):
    r = x_ref.shape[0]
    logits = jnp.dot(x_ref[...], wt_ref[...],
                     preferred_element_type=jnp.float32)
    st = jax.nn.sigmoid(logits.T)

    row = jax.lax.broadcasted_iota(jnp.int32, (NE, r), 0)
    grow = jax.lax.broadcasted_iota(jnp.int32, (NG, r), 0)

    gmax = jnp.concatenate(
        [jnp.max(st[g * GSZ:(g + 1) * GSZ, :], axis=0, keepdims=True)
         for g in range(NG)], axis=0)

    work = gmax
    keep = jnp.zeros((NE, r), jnp.bool_)
    for _ in range(TOPK_G):
        m = jnp.max(work, axis=0, keepdims=True)
        mg = jnp.min(jnp.where(work >= m, grow, 127),
                     axis=0, keepdims=True)
        keep = jnp.logical_or(keep, (row // GSZ) == mg)
        work = jnp.where(grow == mg, NEG, work)

    sm = jnp.where(keep, st, NEG)
    vals, idxs = [], []
    for _ in range(TOPK):
        m = jnp.max(sm, axis=0, keepdims=True)
        mi = jnp.min(jnp.where(sm >= m, row, 127),
                     axis=0, keepdims=True)
        vals.append(m)
        idxs.append(mi)
        sm = jnp.where(row == mi, NEG, sm)

    v = jnp.concatenate(vals, axis=0)
    w_out_ref[...] = v * (SCALE / jnp.sum(v, axis=0, keepdims=True))
    i_out_ref[...] = jnp.concatenate(idxs, axis=0)


@functools.partial(jax.jit, static_argnames=("rows",))
def _route(x, wt, rows=2048):
    grid = (T // rows,)
    return pl.pallas_call(
        _router_body,
        grid=grid,
        in_specs=[
            pl.BlockSpec((rows, DIM), lambda i: (i, 0)),
            pl.BlockSpec((DIM, NE), lambda i: (0, 0)),
        ],
        out_specs=[
            pl.BlockSpec((TOPK, rows), lambda i: (0, i)),
            pl.BlockSpec((TOPK, rows), lambda i: (0, i)),
        ],
        out_shape=[
            jax.ShapeDtypeStruct((TOPK, T), jnp.float32),
            jax.ShapeDtypeStruct((TOPK, T), jnp.int32),
        ],
    )(x, wt)


def kernel(x, weight):
    w, i = _route(x, weight.T)
    return w.T, i.T
